# Initial kernel scaffold; baseline (speedup 1.0000x reference)
import jax
import jax.numpy as jnp
from jax.experimental import pallas as pl


def kernel(x, emb, ln_g, ln_b, Wx_a, bx_a, Wx_b, bx_b, Wv_a, bv_a, Wv_b, bv_b, att_src_0, att_dst_0, att_src_1, att_dst_1, att_src_2, att_dst_2, sem_W, sem_b, sem_q, bn_g, bn_b, W1, b1, W2, b2):
    raise NotImplementedError("write your pallas kernel here")


# speedup vs baseline: 7.6424x; 7.6424x over previous
import jax
import jax.numpy as jnp
from jax import lax
from jax.experimental import pallas as pl

B = 8
SEQ = 64
DH = 64
HEADS = 4
DHH = 16
DOH = 128
N = 4000
K = 8
NEG = -3.0e38

EDGE_SRC = (0, 1, 0)
EDGE_DST = (1, 0, 0)

def _mm(a, b):
    return lax.dot_general(a.astype(jnp.bfloat16), b.astype(jnp.bfloat16),
                           (((1,), (0,)), ((), ())),
                           preferred_element_type=jnp.float32)


def _ln_body(emb_ref, g_ref, b_ref, v_ref):
    e = emb_ref[...]
    mu = jnp.mean(e, axis=-1, keepdims=True)
    var = jnp.mean((e - mu) ** 2, axis=-1, keepdims=True)
    v_ref[...] = (e - mu) / jnp.sqrt(var + 1e-5) * g_ref[...] + b_ref[...]


def _layernorm(emb, g, b):
    return pl.pallas_call(
        _ln_body,
        out_shape=jax.ShapeDtypeStruct((2 * N, DH), jnp.float32),
    )(emb, g.reshape(1, DH), b.reshape(1, DH))


BR = 1000


def _topk_body(vs_ref, vd_ref, nbr_ref):
    vd = vd_ref[...]
    vs = vs_ref[...]
    nd = jnp.sqrt(jnp.sum(vd * vd, axis=1))
    ns = jnp.sqrt(jnp.sum(vs * vs, axis=1))
    outer = ns[:, None] * nd[None, :]
    dot = lax.dot_general(vs.astype(jnp.bfloat16), vd.astype(jnp.bfloat16),
                          (((1,), (1,)), ((), ())),
                          preferred_element_type=jnp.float32)
    sim = dot / outer
    cols = lax.broadcasted_iota(jnp.int32, (BR, N), 1)
    args = []
    for _ in range(K):
        m = jnp.max(sim, axis=1, keepdims=True)
        arg = jnp.min(jnp.where(sim == m, cols, jnp.int32(2**31 - 1)), axis=1)
        args.append(arg)
        sim = jnp.where(cols == arg[:, None], NEG, sim)
    nbr_ref[0] = jnp.stack(args, axis=1)


def _topk(v):
    return pl.pallas_call(
        _topk_body,
        grid=(3, N // BR),
        in_specs=[
            pl.BlockSpec((BR, DH), lambda t, rb: ((t % 2) * (N // BR) + rb, 0)),
            pl.BlockSpec((N, DH), lambda t, rb: (jnp.where(t == 0, 1, 0), 0)),
        ],
        out_specs=pl.BlockSpec((1, BR, K), lambda t, rb: (t, rb, 0)),
        out_shape=jax.ShapeDtypeStruct((3, N, K), jnp.int32),
    )(v, v)


def _proj_body(x_ref, v_ref, wx_ref, bx_ref, wv_ref, bv_ref,
               xp_ref, vp_ref, hs_ref):
    xp = _mm(x_ref[0], wx_ref[0]) + bx_ref[0]
    vp = _mm(v_ref[...], wv_ref[0]) + bv_ref[0]
    xp_ref[0] = xp
    vp_ref[0] = vp
    for h in range(HEADS):
        hs_ref[0, h] = xp[:, h * DHH:(h + 1) * DHH]


def _projections(x16, v, Wx2, bx2, Wv2, bv2):
    return pl.pallas_call(
        _proj_body,
        grid=(2 * B,),
        in_specs=[
            pl.BlockSpec((1, N, SEQ), lambda i: (i, 0, 0)),
            pl.BlockSpec((N, DH), lambda i: (i // B, 0)),
            pl.BlockSpec((1, SEQ, DH), lambda i: (i // B, 0, 0)),
            pl.BlockSpec((1, 1, DH), lambda i: (i // B, 0, 0)),
            pl.BlockSpec((1, DH, DH), lambda i: (i // B, 0, 0)),
            pl.BlockSpec((1, 1, DH), lambda i: (i // B, 0, 0)),
        ],
        out_specs=[
            pl.BlockSpec((1, N, DH), lambda i: (i, 0, 0)),
            pl.BlockSpec((1, N, DH), lambda i: (i, 0, 0)),
            pl.BlockSpec((1, HEADS, N, DHH), lambda i: (i // B, i % B, 0, 0)),
        ],
        out_shape=[
            jax.ShapeDtypeStruct((2 * B, N, DH), jnp.float32),
            jax.ShapeDtypeStruct((2 * B, N, DH), jnp.float32),
            jax.ShapeDtypeStruct((2, B * HEADS, N, DHH), jnp.float32),
        ],
    )(x16, v, Wx2, bx2, Wv2, bv2)


_TSEL = (0, 1, 1, 0, 0, 0)


def _lval_body(xp_ref, att_ref, l_ref):
    xp = xp_ref[0]
    for h in range(HEADS):
        l_ref[0, 0, h] = jnp.sum(
            xp[:, h * DHH:(h + 1) * DHH] * att_ref[0, h, :][None, :], axis=1)


def _lvalues(xp16, att6):
    return pl.pallas_call(
        _lval_body,
        grid=(6, B),
        in_specs=[
            pl.BlockSpec((1, N, DH),
                         lambda q, b: (jnp.where((q == 1) | (q == 2), 1, 0) * B + b,
                                       0, 0)),
            pl.BlockSpec((1, HEADS, DHH), lambda q, b: (q, 0, 0)),
        ],
        out_specs=pl.BlockSpec((1, 1, HEADS, N), lambda q, b: (q, b, 0, 0)),
        out_shape=jax.ShapeDtypeStruct((6, B, HEADS, N), jnp.float32),
    )(xp16, att6)


def _leaky(h, s):
    return jnp.where(h >= 0, h, s * h)


def _edge_aggregate(nbr, L, HS):
    zs, ss = [], []
    i_ids = jnp.repeat(jnp.arange(N), K)
    for t in range(3):
        ls, ld = L[2 * t], L[2 * t + 1]
        M = _leaky(jnp.max(ls, axis=1) + jnp.max(ld, axis=1), 0.2)
        j = nbr[t].reshape(-1)
        logit = _leaky(ls[:, i_ids] + ld[:, j], 0.2)
        e = jnp.exp(logit - M[:, None])
        S = jax.vmap(lambda ee: jax.ops.segment_sum(ee, j, num_segments=N))(e)
        msg = e[:, :, None] * HS[EDGE_SRC[t]][:, i_ids, :]
        Z = jax.vmap(lambda mm: jax.ops.segment_sum(mm, j, num_segments=N))(msg)
        zs.append(Z)
        ss.append(S)
    return jnp.stack(zs), jnp.stack(ss)


def _wpart_body(z_ref, s_ref, sw_ref, sb_ref, sq_ref, w_ref):
    zn = z_ref[0] / (s_ref[0, 0][:, :, None] + 1e-16)
    z64 = jnp.concatenate([zn[h] for h in range(HEADS)], axis=1)
    t = jnp.tanh(_mm(z64, sw_ref[...]) + sb_ref[...])
    tbf = t.astype(jnp.bfloat16).astype(jnp.float32)
    qbf = sq_ref[...].astype(jnp.bfloat16).astype(jnp.float32)
    tq = jnp.sum(tbf * qbf, axis=1)
    w_ref[0, 0] = jnp.broadcast_to(jnp.sum(tq), (DOH,))


def _wparts(ZA, SA, sem_W, sem_b, sem_q):
    return pl.pallas_call(
        _wpart_body,
        grid=(2, B),
        in_specs=[
            pl.BlockSpec((1, HEADS, N, DHH), lambda m, b: (m, b, 0, 0)),
            pl.BlockSpec((1, 1, HEADS, N), lambda m, b: (m, b, 0, 0)),
            pl.BlockSpec((DH, DH), lambda m, b: (0, 0)),
            pl.BlockSpec((1, DH), lambda m, b: (0, 0)),
            pl.BlockSpec((1, DH), lambda m, b: (0, 0)),
        ],
        out_specs=pl.BlockSpec((1, 1, DOH), lambda m, b: (m * B + b, 0, 0)),
        out_shape=jax.ShapeDtypeStruct((2 * B, 1, DOH), jnp.float32),
    )(ZA, SA, sem_W, sem_b.reshape(1, DH), sem_q.reshape(1, DH))


def _p_body(z_ref, s_ref, w_ref, xp_ref, vp_ref, p_ref, st_ref):
    w1 = jnp.sum(w_ref[0:B, 0, :]) / (DOH * B * N)
    w2 = jnp.sum(w_ref[B:2 * B, 0, :]) / (DOH * B * N)
    mw = jnp.maximum(w1, w2)
    e1 = jnp.exp(w1 - mw)
    e2 = jnp.exp(w2 - mw)
    b0 = e1 / (e1 + e2)
    b1 = e2 / (e1 + e2)
    zz = z_ref[0]
    ss = s_ref[0][:, 0]
    zn = zz / (ss[:, :, :, None] + 1e-16)
    z0 = jnp.concatenate([zn[0, h] for h in range(HEADS)], axis=1)
    z1 = jnp.concatenate([zn[1, h] for h in range(HEADS)], axis=1)
    z_comb = b0 * z0 + b1 * z1
    p = z_comb * vp_ref[0] + xp_ref[0]
    p_ref[0] = p
    st_ref[0, 0] = jnp.sum(p, axis=0)
    st_ref[0, 1] = jnp.sum(p * p, axis=0)


def _passemble(Z4, S4, wpart, XP, VP):
    return pl.pallas_call(
        _p_body,
        grid=(2 * B,),
        in_specs=[
            pl.BlockSpec((1, 2, HEADS, N, DHH), lambda i: (i % 2, 0, i // 2, 0, 0)),
            pl.BlockSpec((1, 2, 1, HEADS, N), lambda i: (i % 2, 0, i // 2, 0, 0)),
            pl.BlockSpec((2 * B, 1, DOH), lambda i: (0, 0, 0)),
            pl.BlockSpec((1, N, DH), lambda i: ((i % 2) * B + i // 2, 0, 0)),
            pl.BlockSpec((1, N, DH), lambda i: ((i % 2) * B + i // 2, 0, 0)),
        ],
        out_specs=[
            pl.BlockSpec((1, N, DH), lambda i: (i, 0, 0)),
            pl.BlockSpec((1, 2, DH), lambda i: (i, 0, 0)),
        ],
        out_shape=[
            jax.ShapeDtypeStruct((2 * B, N, DH), jnp.float32),
            jax.ShapeDtypeStruct((2 * B, 2, DH), jnp.float32),
        ],
    )(Z4, S4, wpart, XP, VP)


def _mlp_body(p_ref, st_ref, g_ref, bb_ref, w1_ref, bb1_ref, w2t_ref, b2_ref,
              o_ref):
    nrows = 2.0 * B * N
    mu = jnp.sum(st_ref[:, 0, :], axis=0) / nrows
    ex2 = jnp.sum(st_ref[:, 1, :], axis=0) / nrows
    var = ex2 - mu * mu
    p = (p_ref[0] - mu[None, :]) / jnp.sqrt(var + 1e-5)[None, :]
    p = p * g_ref[...] + bb_ref[...]
    p = _leaky(p, 0.01)
    h = _leaky(_mm(p, w1_ref[...]) + bb1_ref[...], 0.01)
    hbf = h.astype(jnp.bfloat16).astype(jnp.float32)
    w2bf = w2t_ref[...].astype(jnp.bfloat16).astype(jnp.float32)
    o_ref[0, 0] = jnp.sum(hbf * w2bf, axis=1) + b2_ref[0, 0]


def _mlp(P, STAT, bn_g, bn_b, W1, b1, W2, b2):
    return pl.pallas_call(
        _mlp_body,
        grid=(2 * B,),
        in_specs=[
            pl.BlockSpec((1, N, DH), lambda i: (i, 0, 0)),
            pl.BlockSpec((2 * B, 2, DH), lambda i: (0, 0, 0)),
            pl.BlockSpec((1, DH), lambda i: (0, 0)),
            pl.BlockSpec((1, DH), lambda i: (0, 0)),
            pl.BlockSpec((DH, DOH), lambda i: (0, 0)),
            pl.BlockSpec((1, DOH), lambda i: (0, 0)),
            pl.BlockSpec((1, DOH), lambda i: (0, 0)),
            pl.BlockSpec((1, 1), lambda i: (0, 0)),
        ],
        out_specs=pl.BlockSpec((1, 1, N), lambda i: (i, 0, 0)),
        out_shape=jax.ShapeDtypeStruct((2 * B, 1, N), jnp.float32),
    )(P, STAT, bn_g.reshape(1, DH), bn_b.reshape(1, DH), W1,
      b1.reshape(1, DOH), W2.reshape(1, DOH), b2.reshape(1, 1))


def kernel(x, emb, ln_g, ln_b, Wx_a, bx_a, Wx_b, bx_b, Wv_a, bv_a, Wv_b, bv_b,
           att_src_0, att_dst_0, att_src_1, att_dst_1, att_src_2, att_dst_2,
           sem_W, sem_b, sem_q, bn_g, bn_b, W1, b1, W2, b2):
    v = _layernorm(emb, ln_g, ln_b)
    nbr = _topk(v)

    x16 = jnp.concatenate([x[:, :N, :], x[:, N:, :]], axis=0)
    Wx2 = jnp.stack([Wx_a, Wx_b])
    bx2 = jnp.stack([bx_a, bx_b]).reshape(2, 1, DH)
    Wv2 = jnp.stack([Wv_a, Wv_b])
    bv2 = jnp.stack([bv_a, bv_b]).reshape(2, 1, DH)
    XP, VP, HS = _projections(x16, v, Wx2, bx2, Wv2, bv2)

    att6 = jnp.stack([att_src_0, att_dst_0, att_src_1, att_dst_1,
                      att_src_2, att_dst_2])
    L = _lvalues(XP, att6).reshape(6, B * HEADS, N)

    Z, S = _edge_aggregate(nbr, L, HS)

    ZA = Z[1:3]
    SA = S[1:3].reshape(2, B, HEADS, N)
    wpart = _wparts(ZA, SA, sem_W, sem_b, sem_q)

    Z4 = jnp.stack([jnp.stack([Z[1], Z[2]]), jnp.stack([Z[0], Z[0]])])
    S4 = jnp.stack([jnp.stack([S[1], S[2]]),
                    jnp.stack([S[0], S[0]])]).reshape(2, 2, B, HEADS, N)

    P, STAT = _passemble(Z4, S4, wpart, XP, VP)
    O = _mlp(P, STAT, bn_g, bn_b, W1, b1, W2, b2)
    return O.reshape(B, 2, N).reshape(B, 2 * N)


# speedup vs baseline: 19.3309x; 2.5294x over previous
import functools

import jax
import jax.numpy as jnp
from jax import lax
from jax.experimental import pallas as pl
from jax.experimental.pallas import tpu as pltpu
from jax.experimental.pallas import tpu_sc as plsc

B = 8
SEQ = 64
DH = 64
HEADS = 4
DHH = 16
DOH = 128
N = 4000
K = 8
NEG = -3.0e38

EDGE_SRC = (0, 1, 0)
EDGE_DST = (1, 0, 0)

def _mm(a, b):
    return lax.dot_general(a.astype(jnp.bfloat16), b.astype(jnp.bfloat16),
                           (((1,), (0,)), ((), ())),
                           preferred_element_type=jnp.float32)


def _ln_body(emb_ref, g_ref, b_ref, v_ref):
    e = emb_ref[...]
    mu = jnp.mean(e, axis=-1, keepdims=True)
    var = jnp.mean((e - mu) ** 2, axis=-1, keepdims=True)
    v_ref[...] = (e - mu) / jnp.sqrt(var + 1e-5) * g_ref[...] + b_ref[...]


def _layernorm(emb, g, b):
    return pl.pallas_call(
        _ln_body,
        out_shape=jax.ShapeDtypeStruct((2 * N, DH), jnp.float32),
    )(emb, g.reshape(1, DH), b.reshape(1, DH))


BR = 1000


def _topk_body(vs_ref, vd_ref, nbr_ref):
    vd = vd_ref[...]
    vs = vs_ref[...]
    nd = jnp.sqrt(jnp.sum(vd * vd, axis=1))
    ns = jnp.sqrt(jnp.sum(vs * vs, axis=1))
    outer = ns[:, None] * nd[None, :]
    dot = lax.dot_general(vs.astype(jnp.bfloat16), vd.astype(jnp.bfloat16),
                          (((1,), (1,)), ((), ())),
                          preferred_element_type=jnp.float32)
    sim = dot / outer
    cols = lax.broadcasted_iota(jnp.int32, (BR, N), 1)
    args = []
    for _ in range(K):
        m = jnp.max(sim, axis=1, keepdims=True)
        arg = jnp.min(jnp.where(sim == m, cols, jnp.int32(2**31 - 1)), axis=1)
        args.append(arg)
        sim = jnp.where(cols == arg[:, None], NEG, sim)
    nbr_ref[0] = jnp.stack(args, axis=1)


def _topk(v):
    return pl.pallas_call(
        _topk_body,
        grid=(3, N // BR),
        in_specs=[
            pl.BlockSpec((BR, DH), lambda t, rb: ((t % 2) * (N // BR) + rb, 0)),
            pl.BlockSpec((N, DH), lambda t, rb: (jnp.where(t == 0, 1, 0), 0)),
        ],
        out_specs=pl.BlockSpec((1, BR, K), lambda t, rb: (t, rb, 0)),
        out_shape=jax.ShapeDtypeStruct((3, N, K), jnp.int32),
    )(v, v)


def _proj_body(x_ref, v_ref, wx_ref, bx_ref, wv_ref, bv_ref,
               xp_ref, vp_ref, hs_ref):
    xp = _mm(x_ref[0], wx_ref[0]) + bx_ref[0]
    vp = _mm(v_ref[...], wv_ref[0]) + bv_ref[0]
    xp_ref[0] = xp
    vp_ref[0] = vp
    for h in range(HEADS):
        hs_ref[0, h] = xp[:, h * DHH:(h + 1) * DHH]


def _projections(x16, v, Wx2, bx2, Wv2, bv2):
    return pl.pallas_call(
        _proj_body,
        grid=(2 * B,),
        in_specs=[
            pl.BlockSpec((1, N, SEQ), lambda i: (i, 0, 0)),
            pl.BlockSpec((N, DH), lambda i: (i // B, 0)),
            pl.BlockSpec((1, SEQ, DH), lambda i: (i // B, 0, 0)),
            pl.BlockSpec((1, 1, DH), lambda i: (i // B, 0, 0)),
            pl.BlockSpec((1, DH, DH), lambda i: (i // B, 0, 0)),
            pl.BlockSpec((1, 1, DH), lambda i: (i // B, 0, 0)),
        ],
        out_specs=[
            pl.BlockSpec((1, N, DH), lambda i: (i, 0, 0)),
            pl.BlockSpec((1, N, DH), lambda i: (i, 0, 0)),
            pl.BlockSpec((1, HEADS, N, DHH), lambda i: (i // B, i % B, 0, 0)),
        ],
        out_shape=[
            jax.ShapeDtypeStruct((2 * B, N, DH), jnp.float32),
            jax.ShapeDtypeStruct((2 * B, N, DH), jnp.float32),
            jax.ShapeDtypeStruct((2, B * HEADS, N, DHH), jnp.float32),
        ],
    )(x16, v, Wx2, bx2, Wv2, bv2)


_TSEL = (0, 1, 1, 0, 0, 0)


def _lval_body(xp_ref, att_ref, l_ref):
    xp = xp_ref[0]
    for h in range(HEADS):
        l_ref[0, 0, h] = jnp.sum(
            xp[:, h * DHH:(h + 1) * DHH] * att_ref[0, h, :][None, :], axis=1)


def _lvalues(xp16, att6):
    return pl.pallas_call(
        _lval_body,
        grid=(6, B),
        in_specs=[
            pl.BlockSpec((1, N, DH),
                         lambda q, b: (jnp.where((q == 1) | (q == 2), 1, 0) * B + b,
                                       0, 0)),
            pl.BlockSpec((1, HEADS, DHH), lambda q, b: (q, 0, 0)),
        ],
        out_specs=pl.BlockSpec((1, 1, HEADS, N), lambda q, b: (q, b, 0, 0)),
        out_shape=jax.ShapeDtypeStruct((6, B, HEADS, N), jnp.float32),
    )(xp16, att6)


def _leaky(h, s):
    return jnp.where(h >= 0, h, s * h)


_CHUNK = 500
_NBLK = N // _CHUNK
_EC = 1280
_ECN = N * K // _EC


def _ew_body(nbr_ref, ls_ref, ld_ref, e_ref, s_ref):
    c = pl.program_id(1)
    nbrow = nbr_ref[0, 0]
    rows = lax.broadcasted_iota(jnp.int32, (N, _EC), 0)
    oh = (rows == nbrow[None, :]).astype(jnp.float32)
    ld = ld_ref[0]
    ls = ls_ref[0]
    ldg = lax.dot_general(ld, oh, (((1,), (0,)), ((), ())),
                          preferred_element_type=jnp.float32,
                          precision=lax.Precision.HIGHEST)
    nsrc = _EC // K
    eids = lax.broadcasted_iota(jnp.int32, (N, _EC), 1)
    srcids = c * nsrc + eids // K
    ohs = (rows == srcids).astype(jnp.float32)
    lsrep = lax.dot_general(ls, ohs, (((1,), (0,)), ((), ())),
                            preferred_element_type=jnp.float32,
                            precision=lax.Precision.HIGHEST)
    M = jnp.max(ls, axis=1) + jnp.max(ld, axis=1)
    M = jnp.where(M >= 0, M, 0.2 * M)
    lg = lsrep + ldg
    lk = jnp.where(lg >= 0, lg, 0.2 * lg)
    e = jnp.exp(lk - M[:, None])
    e_ref[0] = e
    spart = lax.dot_general(e, oh, (((1,), (1,)), ((), ())),
                            preferred_element_type=jnp.float32,
                            precision=lax.Precision.HIGHEST)

    @pl.when(c == 0)
    def _():
        s_ref[0] = jnp.zeros((B * HEADS, N), jnp.float32)
    s_ref[0] += spart


def _eweights(nbr, L):
    return pl.pallas_call(
        _ew_body,
        grid=(3, _ECN),
        in_specs=[
            pl.BlockSpec((1, 1, _EC), lambda t, c: (t * _ECN + c, 0, 0)),
            pl.BlockSpec((1, B * HEADS, N), lambda t, c: (t, 0, 0)),
            pl.BlockSpec((1, B * HEADS, N), lambda t, c: (t, 0, 0)),
        ],
        out_specs=[
            pl.BlockSpec((1, B * HEADS, _EC), lambda t, c: (t, 0, c)),
            pl.BlockSpec((1, B * HEADS, N), lambda t, c: (t, 0, 0)),
        ],
        out_shape=[
            jax.ShapeDtypeStruct((3, B * HEADS, N * K), jnp.float32),
            jax.ShapeDtypeStruct((3, B * HEADS, N), jnp.float32),
        ],
    )(nbr.reshape(3 * _ECN, 1, _EC), L.reshape(3, 2, B * HEADS, N)[:, 0],
      L.reshape(3, 2, B * HEADS, N)[:, 1])


def _sc_edge_body(nbr_hbm, e_hbm, hs_hbm, z_hbm, hs_v, nbr_v, e_v, zacc):
    ch = lax.axis_index("s") * 2 + lax.axis_index("c")
    zeros16 = jnp.zeros((16,), jnp.float32)

    for t in range(3):
        def zero(i, _):
            zacc[pl.ds(i * 16, 16)] = zeros16
            return 0
        lax.fori_loop(0, N, zero, 0)

        def blk_body(blk, _):
            hs_off = pl.multiple_of(
                (EDGE_SRC[t] * 32 + ch) * (N * DHH) + blk * _CHUNK * DHH,
                _CHUNK * DHH)
            pltpu.sync_copy(hs_hbm.at[pl.ds(hs_off, _CHUNK * DHH)], hs_v)
            pltpu.sync_copy(
                nbr_hbm.at[pl.ds(t * N * K + blk * _CHUNK * K, _CHUNK * K)],
                nbr_v)
            e_off = pl.multiple_of(
                (t * 32 + ch) * (N * K) + blk * _CHUNK * K, _CHUNK * K)
            pltpu.sync_copy(e_hbm.at[pl.ds(e_off, _CHUNK * K)], e_v)

            def echunk(c, _):
                j16 = nbr_v[pl.ds(c * 16, 16)]
                e16 = e_v[pl.ds(c * 16, 16)]
                for l in range(16):
                    j = j16[l]
                    e_s = e16[l]
                    i_loc = c * 2 + (l >> 3)
                    hs_row = hs_v[pl.ds(i_loc * DHH, 16)]
                    zacc[pl.ds(j * DHH, 16)] = (
                        zacc[pl.ds(j * DHH, 16)] + e_s * hs_row)
                return 0
            lax.fori_loop(0, _CHUNK * K // 16, echunk, 0)
            return 0

        lax.fori_loop(0, _NBLK, blk_body, 0)

        z_off = pl.multiple_of((t * 32 + ch) * (N * DHH), N * DHH)
        pltpu.sync_copy(zacc, z_hbm.at[pl.ds(z_off, N * DHH)])


def _sc_edge_aggregate(nbr, E, HS):
    mesh = plsc.VectorSubcoreMesh(core_axis_name="c", subcore_axis_name="s")
    f = pl.kernel(
        _sc_edge_body, mesh=mesh,
        out_type=jax.ShapeDtypeStruct((3 * B * HEADS * N * DHH,), jnp.float32),
        scratch_types=[
            pltpu.VMEM((_CHUNK * DHH,), jnp.float32),
            pltpu.VMEM((_CHUNK * K,), jnp.int32),
            pltpu.VMEM((_CHUNK * K,), jnp.float32),
            pltpu.VMEM((N * DHH,), jnp.float32),
        ],
    )
    Zf = f(nbr.reshape(3 * N * K), E.reshape(3 * B * HEADS * N * K),
           HS.reshape(2 * B * HEADS * N * DHH))
    return Zf.reshape(3, B * HEADS, N, DHH)


def _edge_aggregate(nbr, L, HS):
    zs, ss = [], []
    i_ids = jnp.repeat(jnp.arange(N), K)
    for t in range(3):
        ls, ld = L[2 * t], L[2 * t + 1]
        M = _leaky(jnp.max(ls, axis=1) + jnp.max(ld, axis=1), 0.2)
        j = nbr[t].reshape(-1)
        logit = _leaky(ls[:, i_ids] + ld[:, j], 0.2)
        e = jnp.exp(logit - M[:, None])
        S = jax.vmap(lambda ee: jax.ops.segment_sum(ee, j, num_segments=N))(e)
        msg = e[:, :, None] * HS[EDGE_SRC[t]][:, i_ids, :]
        Z = jax.vmap(lambda mm: jax.ops.segment_sum(mm, j, num_segments=N))(msg)
        zs.append(Z)
        ss.append(S)
    return jnp.stack(zs), jnp.stack(ss)


def _wpart_body(z_ref, s_ref, sw_ref, sb_ref, sq_ref, w_ref):
    zn = z_ref[0] / (s_ref[0, 0][:, :, None] + 1e-16)
    z64 = jnp.concatenate([zn[h] for h in range(HEADS)], axis=1)
    t = jnp.tanh(_mm(z64, sw_ref[...]) + sb_ref[...])
    tbf = t.astype(jnp.bfloat16).astype(jnp.float32)
    qbf = sq_ref[...].astype(jnp.bfloat16).astype(jnp.float32)
    tq = jnp.sum(tbf * qbf, axis=1)
    w_ref[0, 0] = jnp.broadcast_to(jnp.sum(tq), (DOH,))


def _wparts(ZA, SA, sem_W, sem_b, sem_q):
    return pl.pallas_call(
        _wpart_body,
        grid=(2, B),
        in_specs=[
            pl.BlockSpec((1, HEADS, N, DHH), lambda m, b: (m, b, 0, 0)),
            pl.BlockSpec((1, 1, HEADS, N), lambda m, b: (m, b, 0, 0)),
            pl.BlockSpec((DH, DH), lambda m, b: (0, 0)),
            pl.BlockSpec((1, DH), lambda m, b: (0, 0)),
            pl.BlockSpec((1, DH), lambda m, b: (0, 0)),
        ],
        out_specs=pl.BlockSpec((1, 1, DOH), lambda m, b: (m * B + b, 0, 0)),
        out_shape=jax.ShapeDtypeStruct((2 * B, 1, DOH), jnp.float32),
    )(ZA, SA, sem_W, sem_b.reshape(1, DH), sem_q.reshape(1, DH))


def _p_body(z_ref, s_ref, w_ref, xp_ref, vp_ref, p_ref, st_ref):
    w1 = jnp.sum(w_ref[0:B, 0, :]) / (DOH * B * N)
    w2 = jnp.sum(w_ref[B:2 * B, 0, :]) / (DOH * B * N)
    mw = jnp.maximum(w1, w2)
    e1 = jnp.exp(w1 - mw)
    e2 = jnp.exp(w2 - mw)
    b0 = e1 / (e1 + e2)
    b1 = e2 / (e1 + e2)
    zz = z_ref[0]
    ss = s_ref[0][:, 0]
    zn = zz / (ss[:, :, :, None] + 1e-16)
    z0 = jnp.concatenate([zn[0, h] for h in range(HEADS)], axis=1)
    z1 = jnp.concatenate([zn[1, h] for h in range(HEADS)], axis=1)
    z_comb = b0 * z0 + b1 * z1
    p = z_comb * vp_ref[0] + xp_ref[0]
    p_ref[0] = p
    st_ref[0, 0] = jnp.sum(p, axis=0)
    st_ref[0, 1] = jnp.sum(p * p, axis=0)


def _passemble(Z4, S4, wpart, XP, VP):
    return pl.pallas_call(
        _p_body,
        grid=(2 * B,),
        in_specs=[
            pl.BlockSpec((1, 2, HEADS, N, DHH), lambda i: (i % 2, 0, i // 2, 0, 0)),
            pl.BlockSpec((1, 2, 1, HEADS, N), lambda i: (i % 2, 0, i // 2, 0, 0)),
            pl.BlockSpec((2 * B, 1, DOH), lambda i: (0, 0, 0)),
            pl.BlockSpec((1, N, DH), lambda i: ((i % 2) * B + i // 2, 0, 0)),
            pl.BlockSpec((1, N, DH), lambda i: ((i % 2) * B + i // 2, 0, 0)),
        ],
        out_specs=[
            pl.BlockSpec((1, N, DH), lambda i: (i, 0, 0)),
            pl.BlockSpec((1, 2, DH), lambda i: (i, 0, 0)),
        ],
        out_shape=[
            jax.ShapeDtypeStruct((2 * B, N, DH), jnp.float32),
            jax.ShapeDtypeStruct((2 * B, 2, DH), jnp.float32),
        ],
    )(Z4, S4, wpart, XP, VP)


def _mlp_body(p_ref, st_ref, g_ref, bb_ref, w1_ref, bb1_ref, w2t_ref, b2_ref,
              o_ref):
    nrows = 2.0 * B * N
    mu = jnp.sum(st_ref[:, 0, :], axis=0) / nrows
    ex2 = jnp.sum(st_ref[:, 1, :], axis=0) / nrows
    var = ex2 - mu * mu
    p = (p_ref[0] - mu[None, :]) / jnp.sqrt(var + 1e-5)[None, :]
    p = p * g_ref[...] + bb_ref[...]
    p = _leaky(p, 0.01)
    h = _leaky(_mm(p, w1_ref[...]) + bb1_ref[...], 0.01)
    hbf = h.astype(jnp.bfloat16).astype(jnp.float32)
    w2bf = w2t_ref[...].astype(jnp.bfloat16).astype(jnp.float32)
    o_ref[0, 0] = jnp.sum(hbf * w2bf, axis=1) + b2_ref[0, 0]


def _mlp(P, STAT, bn_g, bn_b, W1, b1, W2, b2):
    return pl.pallas_call(
        _mlp_body,
        grid=(2 * B,),
        in_specs=[
            pl.BlockSpec((1, N, DH), lambda i: (i, 0, 0)),
            pl.BlockSpec((2 * B, 2, DH), lambda i: (0, 0, 0)),
            pl.BlockSpec((1, DH), lambda i: (0, 0)),
            pl.BlockSpec((1, DH), lambda i: (0, 0)),
            pl.BlockSpec((DH, DOH), lambda i: (0, 0)),
            pl.BlockSpec((1, DOH), lambda i: (0, 0)),
            pl.BlockSpec((1, DOH), lambda i: (0, 0)),
            pl.BlockSpec((1, 1), lambda i: (0, 0)),
        ],
        out_specs=pl.BlockSpec((1, 1, N), lambda i: (i, 0, 0)),
        out_shape=jax.ShapeDtypeStruct((2 * B, 1, N), jnp.float32),
    )(P, STAT, bn_g.reshape(1, DH), bn_b.reshape(1, DH), W1,
      b1.reshape(1, DOH), W2.reshape(1, DOH), b2.reshape(1, 1))


def kernel(x, emb, ln_g, ln_b, Wx_a, bx_a, Wx_b, bx_b, Wv_a, bv_a, Wv_b, bv_b,
           att_src_0, att_dst_0, att_src_1, att_dst_1, att_src_2, att_dst_2,
           sem_W, sem_b, sem_q, bn_g, bn_b, W1, b1, W2, b2):
    v = _layernorm(emb, ln_g, ln_b)
    nbr = _topk(v)

    x16 = jnp.concatenate([x[:, :N, :], x[:, N:, :]], axis=0)
    Wx2 = jnp.stack([Wx_a, Wx_b])
    bx2 = jnp.stack([bx_a, bx_b]).reshape(2, 1, DH)
    Wv2 = jnp.stack([Wv_a, Wv_b])
    bv2 = jnp.stack([bv_a, bv_b]).reshape(2, 1, DH)
    XP, VP, HS = _projections(x16, v, Wx2, bx2, Wv2, bv2)

    att6 = jnp.stack([att_src_0, att_dst_0, att_src_1, att_dst_1,
                      att_src_2, att_dst_2])
    L = _lvalues(XP, att6).reshape(6, B * HEADS, N)

    E, S = _eweights(nbr, L)
    Z = _sc_edge_aggregate(nbr, E, HS)

    ZA = Z[1:3]
    SA = S[1:3].reshape(2, B, HEADS, N)
    wpart = _wparts(ZA, SA, sem_W, sem_b, sem_q)

    Z4 = jnp.stack([jnp.stack([Z[1], Z[2]]), jnp.stack([Z[0], Z[0]])])
    S4 = jnp.stack([jnp.stack([S[1], S[2]]),
                    jnp.stack([S[0], S[0]])]).reshape(2, 2, B, HEADS, N)

    P, STAT = _passemble(Z4, S4, wpart, XP, VP)
    O = _mlp(P, STAT, bn_g, bn_b, W1, b1, W2, b2)
    return O.reshape(B, 2, N).reshape(B, 2 * N)


# speedup vs baseline: 19.3429x; 1.0006x over previous
import jax
import jax.numpy as jnp
from jax import lax
from jax.experimental import pallas as pl
from jax.experimental.pallas import tpu as pltpu
from jax.experimental.pallas import tpu_sc as plsc

B = 8
SEQ = 64
DH = 64
HEADS = 4
DHH = 16
DOH = 128
N = 4000
K = 8
NEG = -3.0e38

EDGE_SRC = (0, 1, 0)
EDGE_DST = (1, 0, 0)

def _mm(a, b):
    return lax.dot_general(a.astype(jnp.bfloat16), b.astype(jnp.bfloat16),
                           (((1,), (0,)), ((), ())),
                           preferred_element_type=jnp.float32)


def _ln_body(emb_ref, g_ref, b_ref, v_ref):
    e = emb_ref[...]
    mu = jnp.mean(e, axis=-1, keepdims=True)
    var = jnp.mean((e - mu) ** 2, axis=-1, keepdims=True)
    v_ref[...] = (e - mu) / jnp.sqrt(var + 1e-5) * g_ref[...] + b_ref[...]


def _layernorm(emb, g, b):
    return pl.pallas_call(
        _ln_body,
        out_shape=jax.ShapeDtypeStruct((2 * N, DH), jnp.float32),
    )(emb, g.reshape(1, DH), b.reshape(1, DH))


BR = 1000


def _topk_body(vs_ref, vd_ref, nbr_ref):
    vd = vd_ref[...]
    vs = vs_ref[...]
    nd = jnp.sqrt(jnp.sum(vd * vd, axis=1))
    ns = jnp.sqrt(jnp.sum(vs * vs, axis=1))
    outer = ns[:, None] * nd[None, :]
    dot = lax.dot_general(vs.astype(jnp.bfloat16), vd.astype(jnp.bfloat16),
                          (((1,), (1,)), ((), ())),
                          preferred_element_type=jnp.float32)
    sim = dot / outer
    cols = lax.broadcasted_iota(jnp.int32, (BR, N), 1)
    args = []
    for _ in range(K):
        m = jnp.max(sim, axis=1, keepdims=True)
        arg = jnp.min(jnp.where(sim == m, cols, jnp.int32(2**31 - 1)), axis=1)
        args.append(arg)
        sim = jnp.where(cols == arg[:, None], NEG, sim)
    nbr_ref[0] = jnp.stack(args, axis=1)


def _topk(v):
    return pl.pallas_call(
        _topk_body,
        grid=(3, N // BR),
        in_specs=[
            pl.BlockSpec((BR, DH), lambda t, rb: ((t % 2) * (N // BR) + rb, 0)),
            pl.BlockSpec((N, DH), lambda t, rb: (jnp.where(t == 0, 1, 0), 0)),
        ],
        out_specs=pl.BlockSpec((1, BR, K), lambda t, rb: (t, rb, 0)),
        out_shape=jax.ShapeDtypeStruct((3, N, K), jnp.int32),
    )(v, v)


def _proj_body(x_ref, v_ref, wx_ref, bx_ref, wv_ref, bv_ref,
               xp_ref, vp_ref, hs_ref):
    xp = _mm(x_ref[0], wx_ref[0]) + bx_ref[0]
    vp = _mm(v_ref[...], wv_ref[0]) + bv_ref[0]
    xp_ref[0] = xp
    vp_ref[0] = vp
    for h in range(HEADS):
        hs_ref[0, h] = xp[:, h * DHH:(h + 1) * DHH]


def _projections(x16, v, Wx2, bx2, Wv2, bv2):
    return pl.pallas_call(
        _proj_body,
        grid=(2 * B,),
        in_specs=[
            pl.BlockSpec((1, N, SEQ), lambda i: (i, 0, 0)),
            pl.BlockSpec((N, DH), lambda i: (i // B, 0)),
            pl.BlockSpec((1, SEQ, DH), lambda i: (i // B, 0, 0)),
            pl.BlockSpec((1, 1, DH), lambda i: (i // B, 0, 0)),
            pl.BlockSpec((1, DH, DH), lambda i: (i // B, 0, 0)),
            pl.BlockSpec((1, 1, DH), lambda i: (i // B, 0, 0)),
        ],
        out_specs=[
            pl.BlockSpec((1, N, DH), lambda i: (i, 0, 0)),
            pl.BlockSpec((1, N, DH), lambda i: (i, 0, 0)),
            pl.BlockSpec((1, HEADS, N, DHH), lambda i: (i // B, i % B, 0, 0)),
        ],
        out_shape=[
            jax.ShapeDtypeStruct((2 * B, N, DH), jnp.float32),
            jax.ShapeDtypeStruct((2 * B, N, DH), jnp.float32),
            jax.ShapeDtypeStruct((2, B * HEADS, N, DHH), jnp.float32),
        ],
    )(x16, v, Wx2, bx2, Wv2, bv2)


_TSEL = (0, 1, 1, 0, 0, 0)


def _lval_body(xp_ref, att_ref, l_ref):
    xp = xp_ref[0]
    for h in range(HEADS):
        l_ref[0, 0, h] = jnp.sum(
            xp[:, h * DHH:(h + 1) * DHH] * att_ref[0, h, :][None, :], axis=1)


def _lvalues(xp16, att6):
    return pl.pallas_call(
        _lval_body,
        grid=(6, B),
        in_specs=[
            pl.BlockSpec((1, N, DH),
                         lambda q, b: (jnp.where((q == 1) | (q == 2), 1, 0) * B + b,
                                       0, 0)),
            pl.BlockSpec((1, HEADS, DHH), lambda q, b: (q, 0, 0)),
        ],
        out_specs=pl.BlockSpec((1, 1, HEADS, N), lambda q, b: (q, b, 0, 0)),
        out_shape=jax.ShapeDtypeStruct((6, B, HEADS, N), jnp.float32),
    )(xp16, att6)


def _leaky(h, s):
    return jnp.where(h >= 0, h, s * h)


_CHUNK = 500
_NBLK = N // _CHUNK
_EC = 1280
_ECN = N * K // _EC


def _ew_body(nbr_ref, ls_ref, ld_ref, e_ref, s_ref):
    c = pl.program_id(1)
    nbrow = nbr_ref[0, 0]
    rows = lax.broadcasted_iota(jnp.int32, (N, _EC), 0)
    oh = (rows == nbrow[None, :]).astype(jnp.float32)
    ld = ld_ref[0]
    ls = ls_ref[0]
    ldg = lax.dot_general(ld, oh, (((1,), (0,)), ((), ())),
                          preferred_element_type=jnp.float32,
                          precision=lax.Precision.HIGHEST)
    nsrc = _EC // K
    eids = lax.broadcasted_iota(jnp.int32, (N, _EC), 1)
    srcids = c * nsrc + eids // K
    ohs = (rows == srcids).astype(jnp.float32)
    lsrep = lax.dot_general(ls, ohs, (((1,), (0,)), ((), ())),
                            preferred_element_type=jnp.float32,
                            precision=lax.Precision.HIGHEST)
    M = jnp.max(ls, axis=1) + jnp.max(ld, axis=1)
    M = jnp.where(M >= 0, M, 0.2 * M)
    lg = lsrep + ldg
    lk = jnp.where(lg >= 0, lg, 0.2 * lg)
    e = jnp.exp(lk - M[:, None])
    e_ref[0] = e
    spart = lax.dot_general(e, oh, (((1,), (1,)), ((), ())),
                            preferred_element_type=jnp.float32,
                            precision=lax.Precision.HIGHEST)

    @pl.when(c == 0)
    def _():
        s_ref[0] = jnp.zeros((B * HEADS, N), jnp.float32)
    s_ref[0] += spart


def _eweights(nbr, L):
    return pl.pallas_call(
        _ew_body,
        grid=(3, _ECN),
        in_specs=[
            pl.BlockSpec((1, 1, _EC), lambda t, c: (t * _ECN + c, 0, 0)),
            pl.BlockSpec((1, B * HEADS, N), lambda t, c: (t, 0, 0)),
            pl.BlockSpec((1, B * HEADS, N), lambda t, c: (t, 0, 0)),
        ],
        out_specs=[
            pl.BlockSpec((1, B * HEADS, _EC), lambda t, c: (t, 0, c)),
            pl.BlockSpec((1, B * HEADS, N), lambda t, c: (t, 0, 0)),
        ],
        out_shape=[
            jax.ShapeDtypeStruct((3, B * HEADS, N * K), jnp.float32),
            jax.ShapeDtypeStruct((3, B * HEADS, N), jnp.float32),
        ],
    )(nbr.reshape(3 * _ECN, 1, _EC), L.reshape(3, 2, B * HEADS, N)[:, 0],
      L.reshape(3, 2, B * HEADS, N)[:, 1])


def _sc_edge_body(nbr_hbm, e_hbm, hs_hbm, z_hbm, hs_v, nbr_v, e_v, zacc):
    ch = lax.axis_index("s") * 2 + lax.axis_index("c")
    zeros16 = jnp.zeros((16,), jnp.float32)

    for t in range(3):
        def zero(i, _):
            zacc[pl.ds(i * 16, 16)] = zeros16
            return 0
        lax.fori_loop(0, N, zero, 0)

        def blk_body(blk, _):
            hs_off = pl.multiple_of(
                (EDGE_SRC[t] * 32 + ch) * (N * DHH) + blk * _CHUNK * DHH,
                _CHUNK * DHH)
            pltpu.sync_copy(hs_hbm.at[pl.ds(hs_off, _CHUNK * DHH)], hs_v)
            pltpu.sync_copy(
                nbr_hbm.at[pl.ds(t * N * K + blk * _CHUNK * K, _CHUNK * K)],
                nbr_v)
            e_off = pl.multiple_of(
                (t * 32 + ch) * (N * K) + blk * _CHUNK * K, _CHUNK * K)
            pltpu.sync_copy(e_hbm.at[pl.ds(e_off, _CHUNK * K)], e_v)

            def echunk(c, _):
                j16 = nbr_v[pl.ds(c * 16, 16)]
                e16 = e_v[pl.ds(c * 16, 16)]
                for l in range(16):
                    j = j16[l]
                    e_s = e16[l]
                    i_loc = c * 2 + (l >> 3)
                    hs_row = hs_v[pl.ds(i_loc * DHH, 16)]
                    zacc[pl.ds(j * DHH, 16)] = (
                        zacc[pl.ds(j * DHH, 16)] + e_s * hs_row)
                return 0
            lax.fori_loop(0, _CHUNK * K // 16, echunk, 0)
            return 0

        lax.fori_loop(0, _NBLK, blk_body, 0)

        z_off = pl.multiple_of((t * 32 + ch) * (N * DHH), N * DHH)
        pltpu.sync_copy(zacc, z_hbm.at[pl.ds(z_off, N * DHH)])


def _sc_edge_aggregate(nbr, E, HS):
    mesh = plsc.VectorSubcoreMesh(core_axis_name="c", subcore_axis_name="s")
    f = pl.kernel(
        _sc_edge_body, mesh=mesh,
        out_type=jax.ShapeDtypeStruct((3 * B * HEADS * N * DHH,), jnp.float32),
        scratch_types=[
            pltpu.VMEM((_CHUNK * DHH,), jnp.float32),
            pltpu.VMEM((_CHUNK * K,), jnp.int32),
            pltpu.VMEM((_CHUNK * K,), jnp.float32),
            pltpu.VMEM((N * DHH,), jnp.float32),
        ],
    )
    Zf = f(nbr.reshape(3 * N * K), E.reshape(3 * B * HEADS * N * K),
           HS.reshape(2 * B * HEADS * N * DHH))
    return Zf.reshape(3, B * HEADS, N, DHH)


def _wpart_body(z_ref, s_ref, sw_ref, sb_ref, sq_ref, w_ref):
    zn = z_ref[0] / (s_ref[0, 0][:, :, None] + 1e-16)
    z64 = jnp.concatenate([zn[h] for h in range(HEADS)], axis=1)
    t = jnp.tanh(_mm(z64, sw_ref[...]) + sb_ref[...])
    tbf = t.astype(jnp.bfloat16).astype(jnp.float32)
    qbf = sq_ref[...].astype(jnp.bfloat16).astype(jnp.float32)
    tq = jnp.sum(tbf * qbf, axis=1)
    w_ref[0, 0] = jnp.broadcast_to(jnp.sum(tq), (DOH,))


def _wparts(ZA, SA, sem_W, sem_b, sem_q):
    return pl.pallas_call(
        _wpart_body,
        grid=(2, B),
        in_specs=[
            pl.BlockSpec((1, HEADS, N, DHH), lambda m, b: (m, b, 0, 0)),
            pl.BlockSpec((1, 1, HEADS, N), lambda m, b: (m, b, 0, 0)),
            pl.BlockSpec((DH, DH), lambda m, b: (0, 0)),
            pl.BlockSpec((1, DH), lambda m, b: (0, 0)),
            pl.BlockSpec((1, DH), lambda m, b: (0, 0)),
        ],
        out_specs=pl.BlockSpec((1, 1, DOH), lambda m, b: (m * B + b, 0, 0)),
        out_shape=jax.ShapeDtypeStruct((2 * B, 1, DOH), jnp.float32),
    )(ZA, SA, sem_W, sem_b.reshape(1, DH), sem_q.reshape(1, DH))


def _p_body(z_ref, s_ref, w_ref, xp_ref, vp_ref, p_ref, st_ref):
    w1 = jnp.sum(w_ref[0:B, 0, :]) / (DOH * B * N)
    w2 = jnp.sum(w_ref[B:2 * B, 0, :]) / (DOH * B * N)
    mw = jnp.maximum(w1, w2)
    e1 = jnp.exp(w1 - mw)
    e2 = jnp.exp(w2 - mw)
    b0 = e1 / (e1 + e2)
    b1 = e2 / (e1 + e2)
    zz = z_ref[0]
    ss = s_ref[0][:, 0]
    zn = zz / (ss[:, :, :, None] + 1e-16)
    z0 = jnp.concatenate([zn[0, h] for h in range(HEADS)], axis=1)
    z1 = jnp.concatenate([zn[1, h] for h in range(HEADS)], axis=1)
    z_comb = b0 * z0 + b1 * z1
    p = z_comb * vp_ref[0] + xp_ref[0]
    p_ref[0] = p
    st_ref[0, 0] = jnp.sum(p, axis=0)
    st_ref[0, 1] = jnp.sum(p * p, axis=0)


def _passemble(Z4, S4, wpart, XP, VP):
    return pl.pallas_call(
        _p_body,
        grid=(2 * B,),
        in_specs=[
            pl.BlockSpec((1, 2, HEADS, N, DHH), lambda i: (i % 2, 0, i // 2, 0, 0)),
            pl.BlockSpec((1, 2, 1, HEADS, N), lambda i: (i % 2, 0, i // 2, 0, 0)),
            pl.BlockSpec((2 * B, 1, DOH), lambda i: (0, 0, 0)),
            pl.BlockSpec((1, N, DH), lambda i: ((i % 2) * B + i // 2, 0, 0)),
            pl.BlockSpec((1, N, DH), lambda i: ((i % 2) * B + i // 2, 0, 0)),
        ],
        out_specs=[
            pl.BlockSpec((1, N, DH), lambda i: (i, 0, 0)),
            pl.BlockSpec((1, 2, DH), lambda i: (i, 0, 0)),
        ],
        out_shape=[
            jax.ShapeDtypeStruct((2 * B, N, DH), jnp.float32),
            jax.ShapeDtypeStruct((2 * B, 2, DH), jnp.float32),
        ],
    )(Z4, S4, wpart, XP, VP)


def _mlp_body(p_ref, st_ref, g_ref, bb_ref, w1_ref, bb1_ref, w2t_ref, b2_ref,
              o_ref):
    nrows = 2.0 * B * N
    mu = jnp.sum(st_ref[:, 0, :], axis=0) / nrows
    ex2 = jnp.sum(st_ref[:, 1, :], axis=0) / nrows
    var = ex2 - mu * mu
    p = (p_ref[0] - mu[None, :]) / jnp.sqrt(var + 1e-5)[None, :]
    p = p * g_ref[...] + bb_ref[...]
    p = _leaky(p, 0.01)
    h = _leaky(_mm(p, w1_ref[...]) + bb1_ref[...], 0.01)
    hbf = h.astype(jnp.bfloat16).astype(jnp.float32)
    w2bf = w2t_ref[...].astype(jnp.bfloat16).astype(jnp.float32)
    o_ref[0, 0] = jnp.sum(hbf * w2bf, axis=1) + b2_ref[0, 0]


def _mlp(P, STAT, bn_g, bn_b, W1, b1, W2, b2):
    return pl.pallas_call(
        _mlp_body,
        grid=(2 * B,),
        in_specs=[
            pl.BlockSpec((1, N, DH), lambda i: (i, 0, 0)),
            pl.BlockSpec((2 * B, 2, DH), lambda i: (0, 0, 0)),
            pl.BlockSpec((1, DH), lambda i: (0, 0)),
            pl.BlockSpec((1, DH), lambda i: (0, 0)),
            pl.BlockSpec((DH, DOH), lambda i: (0, 0)),
            pl.BlockSpec((1, DOH), lambda i: (0, 0)),
            pl.BlockSpec((1, DOH), lambda i: (0, 0)),
            pl.BlockSpec((1, 1), lambda i: (0, 0)),
        ],
        out_specs=pl.BlockSpec((1, 1, N), lambda i: (i, 0, 0)),
        out_shape=jax.ShapeDtypeStruct((2 * B, 1, N), jnp.float32),
    )(P, STAT, bn_g.reshape(1, DH), bn_b.reshape(1, DH), W1,
      b1.reshape(1, DOH), W2.reshape(1, DOH), b2.reshape(1, 1))


def kernel(x, emb, ln_g, ln_b, Wx_a, bx_a, Wx_b, bx_b, Wv_a, bv_a, Wv_b, bv_b,
           att_src_0, att_dst_0, att_src_1, att_dst_1, att_src_2, att_dst_2,
           sem_W, sem_b, sem_q, bn_g, bn_b, W1, b1, W2, b2):
    v = _layernorm(emb, ln_g, ln_b)
    nbr = _topk(v)

    x16 = jnp.concatenate([x[:, :N, :], x[:, N:, :]], axis=0)
    Wx2 = jnp.stack([Wx_a, Wx_b])
    bx2 = jnp.stack([bx_a, bx_b]).reshape(2, 1, DH)
    Wv2 = jnp.stack([Wv_a, Wv_b])
    bv2 = jnp.stack([bv_a, bv_b]).reshape(2, 1, DH)
    XP, VP, HS = _projections(x16, v, Wx2, bx2, Wv2, bv2)

    att6 = jnp.stack([att_src_0, att_dst_0, att_src_1, att_dst_1,
                      att_src_2, att_dst_2])
    L = _lvalues(XP, att6).reshape(6, B * HEADS, N)

    E, S = _eweights(nbr, L)
    Z = _sc_edge_aggregate(nbr, E, HS)

    ZA = Z[1:3]
    SA = S[1:3].reshape(2, B, HEADS, N)
    wpart = _wparts(ZA, SA, sem_W, sem_b, sem_q)

    Z4 = jnp.stack([jnp.stack([Z[1], Z[2]]), jnp.stack([Z[0], Z[0]])])
    S4 = jnp.stack([jnp.stack([S[1], S[2]]),
                    jnp.stack([S[0], S[0]])]).reshape(2, 2, B, HEADS, N)

    P, STAT = _passemble(Z4, S4, wpart, XP, VP)
    O = _mlp(P, STAT, bn_g, bn_b, W1, b1, W2, b2)
    return O.reshape(B, 2, N).reshape(B, 2 * N)
